# baseline (speedup 1.0000x reference)
import jax
import jax.numpy as jnp
from jax.experimental import pallas as pl
from jax.experimental.pallas import tpu as pltpu

_B = 131072


def _body_a(m_ref, md_ref, s_ref, mo_ref, so_ref):
    mo_ref[...] = m_ref[...] + md_ref[...]
    so_ref[...] = s_ref[...]


def _body_b(r_ref, rd_ref, dummy_ref, ro_ref):
    ro_ref[...] = r_ref[...] + rd_ref[...]


def kernel(means, scales, rot, means_def, rot_def):
    n = means.shape[0]
    g = pl.cdiv(n, _B)
    bs3 = pl.BlockSpec((3, _B), lambda i: (0, i))
    bs4 = pl.BlockSpec((4, _B), lambda i: (0, i))
    mo_t, so_t = pl.pallas_call(
        _body_a,
        grid=(g,),
        in_specs=[bs3, bs3, bs3],
        out_specs=[bs3, bs3],
        out_shape=[
            jax.ShapeDtypeStruct((3, n), means.dtype),
            jax.ShapeDtypeStruct((3, n), scales.dtype),
        ],
        compiler_params=pltpu.CompilerParams(vmem_limit_bytes=20 * 1024 * 1024),
    )(means.T, means_def.T, scales.T)
    dummy = jax.lax.slice(mo_t, (0, 0), (3, 128))
    ro_t = pl.pallas_call(
        _body_b,
        grid=(g,),
        in_specs=[bs4, bs4, pl.BlockSpec((3, 128), lambda i: (0, 0))],
        out_specs=bs4,
        out_shape=jax.ShapeDtypeStruct((4, n), rot.dtype),
        compiler_params=pltpu.CompilerParams(vmem_limit_bytes=16 * 1024 * 1024),
    )(rot.T, rot_def.T, dummy)
    return (mo_t.T, so_t.T, ro_t.T)


# speedup vs baseline: 1.1727x; 1.1727x over previous
import jax
import jax.numpy as jnp
from jax.experimental import pallas as pl
from jax.experimental.pallas import tpu as pltpu


def _body(m_ref, md_ref, r_ref, rd_ref, s_ref, mo_ref, ro_ref, so_ref):
    mo_ref[...] = m_ref[...] + md_ref[...]
    ro_ref[...] = r_ref[...] + rd_ref[...]
    so_ref[...] = s_ref[...]


def kernel(means, scales, rot, means_def, rot_def):
    n = means.shape[0]
    B = 131072
    g = pl.cdiv(n, B)
    bs3 = pl.BlockSpec((3, B), lambda i: (0, i))
    bs4 = pl.BlockSpec((4, B), lambda i: (0, i))
    mo_t, ro_t, so_t = pl.pallas_call(
        _body,
        grid=(g,),
        in_specs=[bs3, bs3, bs4, bs4, bs3],
        out_specs=[bs3, bs4, bs3],
        out_shape=[
            jax.ShapeDtypeStruct((3, n), means.dtype),
            jax.ShapeDtypeStruct((4, n), rot.dtype),
            jax.ShapeDtypeStruct((3, n), scales.dtype),
        ],
        compiler_params=pltpu.CompilerParams(vmem_limit_bytes=56 * 1024 * 1024),
    )(means.T, means_def.T, rot.T, rot_def.T, scales.T)
    return (mo_t.T, so_t.T, ro_t.T)
